# baseline (speedup 1.0000x reference)
import jax
import jax.numpy as jnp
from jax.experimental import pallas as pl
from jax.experimental.pallas import tpu as pltpu

_SIGMA = 0.5
_INV = 1.0 / (_SIGMA * (2.0 ** 0.5))
_NC = 16
_NBUF = 6
_MBUF = 2


def _body(x_hbm, mu_hbm, nz_hbm, out_ref, l0_ref,
          xb, nb, mb, acc_s, xs, ns, ms):
    batch = x_hbm.shape[0]
    nchunk = x_hbm.shape[1]
    steps = batch * nchunk
    t = pl.program_id(0)
    b = jax.lax.rem(t, batch)
    c = jax.lax.div(t, batch)
    slot = jax.lax.rem(t, _NBUF)
    mslot = jax.lax.rem(c, _MBUF)

    def in_copy(tt, s):
        bb = jax.lax.rem(tt, batch)
        cc = jax.lax.div(tt, batch)
        pltpu.make_async_copy(x_hbm.at[bb, cc], xb.at[s], xs.at[s]).start()
        pltpu.make_async_copy(nz_hbm.at[bb, cc], nb.at[s], ns.at[s]).start()

    def mu_copy(cc):
        s = jax.lax.rem(cc, _MBUF)
        pltpu.make_async_copy(mu_hbm.at[cc], mb.at[s], ms.at[s]).start()

    @pl.when(t == 0)
    def _prologue():
        acc_s[0] = 0.0
        mu_copy(0)
        in_copy(0, 0)
        mu_copy(1)
        for k in range(1, _NBUF):
            in_copy(k, k)

    pltpu.make_async_copy(x_hbm.at[0, 0], xb.at[slot], xs.at[slot]).wait()
    pltpu.make_async_copy(nz_hbm.at[0, 0], nb.at[slot], ns.at[slot]).wait()

    @pl.when(b == 0)
    def _wait_mu():
        pltpu.make_async_copy(mu_hbm.at[0], mb.at[mslot], ms.at[mslot]).wait()

    mu = mb[mslot]
    gate = jnp.clip(mu + _SIGMA * nb[slot], 0.0, 1.0)
    out_ref[0, 0] = xb[slot] * gate

    @pl.when(b == 0)
    def _erf():
        p = 0.5 * (1.0 + jax.lax.erf(mu * _INV))
        acc_s[0] += jnp.sum(p)

    @pl.when(t + _NBUF < steps)
    def _refill():
        in_copy(t + _NBUF, slot)

    @pl.when((b == batch - 1) & (c + _MBUF < nchunk))
    def _mu_refill():
        mu_copy(c + _MBUF)

    @pl.when(t == steps - 1)
    def _final():
        l0_ref[...] = jnp.broadcast_to(acc_s[0], (1, 128))


@jax.jit
def kernel(input_tensor, mu, noise):
    b = input_tensor.shape[0]
    n = mu.shape[0]
    rows = n // (_NC * 1024)
    x4 = input_tensor.reshape(b, _NC, rows, 1024)
    nz4 = noise.reshape(b, _NC, rows, 1024)
    mu3 = mu.reshape(_NC, rows, 1024)
    steps = b * _NC
    gated, l0 = pl.pallas_call(
        _body,
        grid=(steps,),
        in_specs=[
            pl.BlockSpec(memory_space=pl.ANY),
            pl.BlockSpec(memory_space=pl.ANY),
            pl.BlockSpec(memory_space=pl.ANY),
        ],
        out_specs=[
            pl.BlockSpec((1, 1, rows, 1024),
                         lambda t: (t % b, t // b, 0, 0)),
            pl.BlockSpec((1, 128), lambda t: (0, 0)),
        ],
        out_shape=[
            jax.ShapeDtypeStruct((b, _NC, rows, 1024), jnp.float32),
            jax.ShapeDtypeStruct((1, 128), jnp.float32),
        ],
        scratch_shapes=[
            pltpu.VMEM((_NBUF, rows, 1024), jnp.float32),
            pltpu.VMEM((_NBUF, rows, 1024), jnp.float32),
            pltpu.VMEM((_MBUF, rows, 1024), jnp.float32),
            pltpu.SMEM((1,), jnp.float32),
            pltpu.SemaphoreType.DMA((_NBUF,)),
            pltpu.SemaphoreType.DMA((_NBUF,)),
            pltpu.SemaphoreType.DMA((_MBUF,)),
        ],
    )(x4, mu3, nz4)
    return gated.reshape(input_tensor.shape), l0[0, 0]
